# baseline (speedup 1.0000x reference)
import functools

import jax
import jax.numpy as jnp
from jax import lax
from jax.experimental import pallas as pl
from jax.experimental.pallas import tpu as pltpu
from jax.experimental.pallas import tpu_sc as plsc

_B = 16384
_L = 20
_D1 = 3
_D2 = 32
_V1 = 100
_V2 = 200
_OUT = _L * _D1 + _D2
_NC = 2
_NS = 16
_NW = _NC * _NS
_R = _B // _NW
_C = 128
_NCH = _R // _C
_GC = _C // 16

_mesh = plsc.VectorSubcoreMesh(core_axis_name="c", subcore_axis_name="s")


def _body(idx1_hbm, idx2_hbm, t1_hbm, t2_hbm, out_hbm,
          idx1_v, idx2_v, t1_v, t2_v, out_v, sem):
    wid = lax.axis_index("s") * _NC + lax.axis_index("c")
    base = wid * _R

    ct1 = pltpu.async_copy(t1_hbm, t1_v, sem)
    ct2 = pltpu.async_copy(t2_hbm, t2_v, sem)
    ct1.wait()
    ct2.wait()

    inv_l = jnp.float32(1.0 / _L)

    for k in range(_NCH):
        cbase = base + k * _C
        c1 = pltpu.async_copy(idx1_hbm.at[pl.ds(cbase, _C)], idx1_v, sem)
        c2 = pltpu.async_copy(idx2_hbm.at[pl.ds(cbase, _C)], idx2_v, sem)
        c1.wait()
        c2.wait()

        def group(g, carry):
            rows = g * 16 + lax.iota(jnp.int32, 16)
            for l in range(_L):
                lv = jnp.full((16,), l, jnp.int32)
                iv = plsc.load_gather(idx1_v, [rows, lv])
                for c in range(_D1):
                    cv = jnp.full((16,), c, jnp.int32)
                    vals = plsc.load_gather(t1_v, [iv, cv])
                    plsc.store_scatter(
                        out_v,
                        [rows, jnp.full((16,), l * _D1 + c, jnp.int32)], vals)
            acc = [jnp.zeros((16,), jnp.float32) for _ in range(_D2)]
            for l in range(_L):
                lv = jnp.full((16,), l, jnp.int32)
                iv = plsc.load_gather(idx2_v, [rows, lv])
                for d in range(_D2):
                    dv = jnp.full((16,), d, jnp.int32)
                    acc[d] = acc[d] + plsc.load_gather(t2_v, [iv, dv])
            for d in range(_D2):
                plsc.store_scatter(
                    out_v,
                    [rows, jnp.full((16,), _L * _D1 + d, jnp.int32)],
                    acc[d] * inv_l)
            return carry

        lax.fori_loop(0, _GC, group, 0)
        pltpu.sync_copy(out_v, out_hbm.at[pl.ds(cbase, _C)])


_run = functools.partial(
    pl.kernel,
    out_type=jax.ShapeDtypeStruct((_B, _OUT), jnp.float32),
    mesh=_mesh,
    compiler_params=pltpu.CompilerParams(needs_layout_passes=False),
    scratch_types=[
        pltpu.VMEM((_C, _L), jnp.int32),
        pltpu.VMEM((_C, _L), jnp.int32),
        pltpu.VMEM((_V1, _D1), jnp.float32),
        pltpu.VMEM((_V2, _D2), jnp.float32),
        pltpu.VMEM((_C, _OUT), jnp.float32),
        pltpu.SemaphoreType.DMA,
    ],
)(_body)


@jax.jit
def kernel(idx_emb1, idx_embbag1, emb1_w, embbag1_w):
    return _run(idx_emb1, idx_embbag1, emb1_w, embbag1_w)


# speedup vs baseline: 3.5191x; 3.5191x over previous
import functools

import jax
import jax.numpy as jnp
from jax import lax
from jax.experimental import pallas as pl
from jax.experimental.pallas import tpu as pltpu
from jax.experimental.pallas import tpu_sc as plsc

_B = 16384
_L = 20
_D1 = 3
_D2 = 32
_V1 = 100
_V2 = 200
_OUT = _L * _D1 + _D2
_NC = 2
_NS = 16
_NW = _NC * _NS
_R = _B // _NW
_G = _R // 16
_T2OFF = _V1 * _D1

_mesh = plsc.VectorSubcoreMesh(core_axis_name="c", subcore_axis_name="s")


def _body(idx_hbm, tab_hbm, out_hbm, idx1_v, idx2_v, tab_v, out_v, sem):
    wid = lax.axis_index("s") * _NC + lax.axis_index("c")
    base = wid * _R

    c1 = pltpu.async_copy(idx_hbm.at[pl.ds(base * _L, _R * _L)], idx1_v, sem)
    c2 = pltpu.async_copy(
        idx_hbm.at[pl.ds(_B * _L + base * _L, _R * _L)], idx2_v, sem)
    c3 = pltpu.async_copy(tab_hbm, tab_v, sem)
    c1.wait()
    c2.wait()
    c3.wait()

    inv_l = jnp.float32(1.0 / _L)

    @functools.partial(plsc.parallel_loop, 0, _G, unroll=1)
    def group(g):
        rows = g * 16 + lax.iota(jnp.int32, 16)
        ibase = rows * _L
        obase = rows * _OUT
        for l in range(_L):
            iv = plsc.load_gather(idx1_v, [ibase + l])
            for c in range(_D1):
                vals = plsc.load_gather(tab_v, [iv * _D1 + c])
                plsc.store_scatter(out_v, [obase + (l * _D1 + c)], vals)
        acc = [jnp.zeros((16,), jnp.float32) for _ in range(_D2)]
        for l in range(_L):
            iv = plsc.load_gather(idx2_v, [ibase + l])
            ivd = iv * _D2 + _T2OFF
            for d in range(_D2):
                acc[d] = acc[d] + plsc.load_gather(tab_v, [ivd + d])
        for d in range(_D2):
            plsc.store_scatter(out_v, [obase + (_L * _D1 + d)], acc[d] * inv_l)

    pltpu.sync_copy(out_v, out_hbm.at[pl.ds(base * _OUT, _R * _OUT)])


_run = functools.partial(
    pl.kernel,
    out_type=jax.ShapeDtypeStruct((_B * _OUT,), jnp.float32),
    mesh=_mesh,
    compiler_params=pltpu.CompilerParams(needs_layout_passes=False),
    scratch_types=[
        pltpu.VMEM((_R * _L,), jnp.int32),
        pltpu.VMEM((_R * _L,), jnp.int32),
        pltpu.VMEM((_V1 * _D1 + _V2 * _D2,), jnp.float32),
        pltpu.VMEM((_R * _OUT,), jnp.float32),
        pltpu.SemaphoreType.DMA,
    ],
)(_body)


@jax.jit
def kernel(idx_emb1, idx_embbag1, emb1_w, embbag1_w):
    idx_all = jnp.concatenate(
        [idx_emb1.astype(jnp.int32).reshape(-1),
         idx_embbag1.astype(jnp.int32).reshape(-1)])
    tab_all = jnp.concatenate([emb1_w.reshape(-1), embbag1_w.reshape(-1)])
    out = _run(idx_all, tab_all)
    return out.reshape(_B, _OUT)


# speedup vs baseline: 3.7701x; 1.0713x over previous
import functools

import jax
import jax.numpy as jnp
from jax import lax
from jax.experimental import pallas as pl
from jax.experimental.pallas import tpu as pltpu
from jax.experimental.pallas import tpu_sc as plsc

_B = 16384
_L = 20
_D1 = 3
_D2 = 32
_V1 = 100
_V2 = 200
_OUT = _L * _D1 + _D2
_NC = 2
_NS = 16
_NW = _NC * _NS
_R = _B // _NW
_G = _R // 16
_C = 128
_NCH = _R // _C
_GC = _C // 16

_mesh = plsc.VectorSubcoreMesh(core_axis_name="c", subcore_axis_name="s")


def _body(idx1_hbm, idx2_hbm, t1_hbm, t2_hbm, out_hbm,
          sta_v, stb_v, idx1_v, idx2_v, t1_v, t2_v, out_v, sem):
    wid = lax.axis_index("s") * _NC + lax.axis_index("c")
    base = wid * _R

    ct1 = pltpu.async_copy(t1_hbm, t1_v, sem)
    ct2 = pltpu.async_copy(t2_hbm, t2_v, sem)

    for k in range(_NCH):
        c1 = pltpu.async_copy(idx1_hbm.at[pl.ds(base + k * _C, _C)],
                              sta_v, sem)
        c2 = pltpu.async_copy(idx2_hbm.at[pl.ds(base + k * _C, _C)],
                              stb_v, sem)
        c1.wait()
        c2.wait()

        def repack(g, carry):
            rows = g * 16 + lax.iota(jnp.int32, 16)
            fbase = (k * _C + g * 16 + lax.iota(jnp.int32, 16)) * _L
            v1 = [plsc.load_gather(sta_v,
                                   [rows, jnp.full((16,), l, jnp.int32)])
                  for l in range(_L)]
            for l in range(_L):
                plsc.store_scatter(idx1_v, [fbase + l], v1[l])
            v2 = [plsc.load_gather(stb_v,
                                   [rows, jnp.full((16,), l, jnp.int32)])
                  for l in range(_L)]
            for l in range(_L):
                plsc.store_scatter(idx2_v, [fbase + l], v2[l])
            return carry

        lax.fori_loop(0, _GC, repack, 0)

    ct1.wait()
    ct2.wait()

    inv_l = jnp.float32(1.0 / _L)

    @functools.partial(plsc.parallel_loop, 0, _G, unroll=1)
    def group(g):
        rows = g * 16 + lax.iota(jnp.int32, 16)
        ibase = rows * _L
        obase = rows * _OUT
        for l in range(_L):
            iv = plsc.load_gather(idx1_v, [ibase + l])
            for c in range(_D1):
                vals = plsc.load_gather(t1_v, [iv * _D1 + c])
                plsc.store_scatter(out_v, [obase + (l * _D1 + c)], vals)
        acc = [jnp.zeros((16,), jnp.float32) for _ in range(_D2)]
        for l in range(_L):
            iv = plsc.load_gather(idx2_v, [ibase + l])
            ivd = iv * _D2
            for d in range(_D2):
                acc[d] = acc[d] + plsc.load_gather(t2_v, [ivd + d])
        for d in range(_D2):
            plsc.store_scatter(out_v, [obase + (_L * _D1 + d)], acc[d] * inv_l)

    pltpu.sync_copy(out_v, out_hbm.at[pl.ds(base * _OUT, _R * _OUT)])


_run = functools.partial(
    pl.kernel,
    out_type=jax.ShapeDtypeStruct((_B * _OUT,), jnp.float32),
    mesh=_mesh,
    compiler_params=pltpu.CompilerParams(needs_layout_passes=False),
    scratch_types=[
        pltpu.VMEM((_C, _L), jnp.int32),
        pltpu.VMEM((_C, _L), jnp.int32),
        pltpu.VMEM((_R * _L,), jnp.int32),
        pltpu.VMEM((_R * _L,), jnp.int32),
        pltpu.VMEM((_V1 * _D1,), jnp.float32),
        pltpu.VMEM((_V2 * _D2,), jnp.float32),
        pltpu.VMEM((_R * _OUT,), jnp.float32),
        pltpu.SemaphoreType.DMA,
    ],
)(_body)


@jax.jit
def kernel(idx_emb1, idx_embbag1, emb1_w, embbag1_w):
    out = _run(idx_emb1, idx_embbag1, emb1_w.reshape(-1),
               embbag1_w.reshape(-1))
    return out.reshape(_B, _OUT)
